# baseline (speedup 1.0000x reference)
import functools

import jax
import jax.numpy as jnp
from jax import lax
from jax.experimental import pallas as pl
from jax.experimental.pallas import tpu as pltpu
from jax.experimental.pallas import tpu_sc as plsc

NC = 2
NS = 16
NW = NC * NS

BATCH = 16384
GMF_DIM = 32
MLP_EMB_DIM = 64
B_PER_W = BATCH // NW


def _sc_gather_body(uid_hbm, iid_hbm, mlp_P, mlp_Q, gmf_P, gmf_Q,
                    o_pu, o_qi, o_gu, o_gv,
                    uidx, iidx, bpu, bqi, bgu, bgv, sem):
    wid = lax.axis_index("s") * NC + lax.axis_index("c")
    base = wid * B_PER_W
    pltpu.sync_copy(uid_hbm.at[pl.ds(base, B_PER_W)], uidx)
    pltpu.sync_copy(iid_hbm.at[pl.ds(base, B_PER_W)], iidx)
    cp1 = pltpu.async_copy(mlp_P.at[uidx], bpu, sem)
    cp2 = pltpu.async_copy(mlp_Q.at[iidx], bqi, sem)
    cp3 = pltpu.async_copy(gmf_P.at[uidx], bgu, sem)
    cp4 = pltpu.async_copy(gmf_Q.at[iidx], bgv, sem)
    cp1.wait()
    cp2.wait()
    cp3.wait()
    cp4.wait()
    pltpu.sync_copy(bpu, o_pu.at[pl.ds(base, B_PER_W)])
    pltpu.sync_copy(bqi, o_qi.at[pl.ds(base, B_PER_W)])
    pltpu.sync_copy(bgu, o_gu.at[pl.ds(base, B_PER_W)])
    pltpu.sync_copy(bgv, o_gv.at[pl.ds(base, B_PER_W)])


_sc_gather = functools.partial(
    pl.kernel,
    out_type=(
        jax.ShapeDtypeStruct((BATCH, MLP_EMB_DIM), jnp.float32),
        jax.ShapeDtypeStruct((BATCH, MLP_EMB_DIM), jnp.float32),
        jax.ShapeDtypeStruct((BATCH, GMF_DIM), jnp.float32),
        jax.ShapeDtypeStruct((BATCH, GMF_DIM), jnp.float32),
    ),
    mesh=plsc.VectorSubcoreMesh(core_axis_name="c", subcore_axis_name="s"),
    compiler_params=pltpu.CompilerParams(use_tc_tiling_on_sc=False),
    scratch_types=[
        pltpu.VMEM((B_PER_W,), jnp.int32),
        pltpu.VMEM((B_PER_W,), jnp.int32),
        pltpu.VMEM((B_PER_W, MLP_EMB_DIM), jnp.float32),
        pltpu.VMEM((B_PER_W, MLP_EMB_DIM), jnp.float32),
        pltpu.VMEM((B_PER_W, GMF_DIM), jnp.float32),
        pltpu.VMEM((B_PER_W, GMF_DIM), jnp.float32),
        pltpu.SemaphoreType.DMA,
    ],
)(_sc_gather_body)


TC_BLOCK = 2048


def _tc_dense_body(pu_ref, qi_ref, gu_ref, gv_ref,
                   W1a_ref, W1b_ref, b1_ref, W2_ref, b2_ref,
                   wg_ref, wm_ref, bo_ref, out_ref):
    h1 = jnp.maximum(
        jnp.dot(pu_ref[...], W1a_ref[...], preferred_element_type=jnp.float32)
        + jnp.dot(qi_ref[...], W1b_ref[...], preferred_element_type=jnp.float32)
        + b1_ref[...], 0.0)
    h2 = jnp.maximum(
        jnp.dot(h1, W2_ref[...], preferred_element_type=jnp.float32)
        + b2_ref[...], 0.0)
    gmf = gu_ref[...] * gv_ref[...]
    out = (jnp.sum(gmf * wg_ref[...], axis=-1, keepdims=True)
           + jnp.sum(h2 * wm_ref[...], axis=-1, keepdims=True)
           + bo_ref[0, 0])
    out_ref[...] = out


def kernel(user_id, item_id, gmf_P, gmf_Q, mlp_P, mlp_Q, W1, b1, W2, b2, W_out, b_out):
    pu, qi, gu, gv = _sc_gather(user_id, item_id, mlp_P, mlp_Q, gmf_P, gmf_Q)

    W1a = W1[:, :MLP_EMB_DIM].T
    W1b = W1[:, MLP_EMB_DIM:].T
    W2t = W2.T
    wg = W_out[:, :GMF_DIM]
    wm = W_out[:, GMF_DIM:]
    bo = b_out.reshape(1, 1)

    grid = (BATCH // TC_BLOCK,)
    out = pl.pallas_call(
        _tc_dense_body,
        grid=grid,
        in_specs=[
            pl.BlockSpec((TC_BLOCK, MLP_EMB_DIM), lambda i: (i, 0)),
            pl.BlockSpec((TC_BLOCK, MLP_EMB_DIM), lambda i: (i, 0)),
            pl.BlockSpec((TC_BLOCK, GMF_DIM), lambda i: (i, 0)),
            pl.BlockSpec((TC_BLOCK, GMF_DIM), lambda i: (i, 0)),
            pl.BlockSpec((MLP_EMB_DIM, MLP_EMB_DIM), lambda i: (0, 0)),
            pl.BlockSpec((MLP_EMB_DIM, MLP_EMB_DIM), lambda i: (0, 0)),
            pl.BlockSpec((1, MLP_EMB_DIM), lambda i: (0, 0)),
            pl.BlockSpec((MLP_EMB_DIM, GMF_DIM), lambda i: (0, 0)),
            pl.BlockSpec((1, GMF_DIM), lambda i: (0, 0)),
            pl.BlockSpec((1, GMF_DIM), lambda i: (0, 0)),
            pl.BlockSpec((1, GMF_DIM), lambda i: (0, 0)),
            pl.BlockSpec((1, 1), lambda i: (0, 0), memory_space=pltpu.SMEM),
        ],
        out_specs=pl.BlockSpec((TC_BLOCK, 1), lambda i: (i, 0)),
        out_shape=jax.ShapeDtypeStruct((BATCH, 1), jnp.float32),
    )(pu, qi, gu, gv, W1a, W1b, b1.reshape(1, -1), W2t, b2.reshape(1, -1),
      wg, wm, bo)
    return out
